# baseline (speedup 1.0000x reference)
import functools

import jax
import jax.numpy as jnp
from jax import lax
from jax.experimental import pallas as pl
from jax.experimental.pallas import tpu as pltpu
from jax.experimental.pallas import tpu_sc as plsc

NEXP = 8
BT = 256
NBP = 32
NW = 32
SUB = 32
HALF = 32


def _route_kernel(x_ref, wg_ref, pos_ref, w_ref, be_ref, act_ref):
    scoresT = lax.dot_general(wg_ref[...], x_ref[...],
                              (((1,), (1,)), ((), ())),
                              preferred_element_type=jnp.float32)
    e_, s_ = scoresT.shape
    sub = lax.broadcasted_iota(jnp.int32, (e_, s_), 0)
    v1 = jnp.max(scoresT, axis=0, keepdims=True)
    i1 = jnp.min(jnp.where(scoresT == v1, sub, e_), axis=0, keepdims=True)
    masked = jnp.where(sub == i1, -jnp.inf, scoresT)
    v2 = jnp.max(masked, axis=0, keepdims=True)
    i2 = jnp.min(jnp.where(masked == v2, sub, e_), axis=0, keepdims=True)
    w1 = 1.0 / (1.0 + jnp.exp(v2 - v1))
    w_ref[...] = jnp.concatenate([w1, 1.0 - w1], axis=1)

    oh = jnp.concatenate([sub == i1, sub == i2], axis=1).astype(jnp.int32)
    c = oh
    sh = 1
    while sh < 2 * s_:
        c = c + jnp.concatenate(
            [jnp.zeros((e_, sh), jnp.int32), c[:, :-sh]], axis=1)
        sh *= 2
    excl = c - oh
    counts = c[:, -1:]
    pc = ((counts + BT - 1) // BT) * BT
    oc = pc
    sh = 1
    while sh < e_:
        oc = oc + jnp.concatenate(
            [jnp.zeros((sh, 1), jnp.int32), oc[:-sh]], axis=0)
        sh *= 2
    off = oc - pc
    end = oc
    pos_ref[...] = jnp.sum(oh * (off + excl), axis=0, keepdims=True)

    starts = lax.broadcasted_iota(jnp.int32, (1, NBP), 1) * BT
    nbefore = jnp.sum((starts >= end).astype(jnp.int32), axis=0, keepdims=True)
    be_ref[...] = jnp.minimum(nbefore, e_ - 1)
    act_ref[...] = (starts < end[e_ - 1:e_, :]).astype(jnp.int32)


def _ffn_kernel(be_ref, act_ref, xs_ref, w1_ref, w3_ref, w2_ref, ws_ref, o_ref):
    b = pl.program_id(0)

    @pl.when(act_ref[b] == 1)
    def _():
        xb = xs_ref[...]
        g = lax.dot_general(xb, w1_ref[0], (((1,), (1,)), ((), ())),
                            preferred_element_type=jnp.float32)
        u = lax.dot_general(xb, w3_ref[0], (((1,), (1,)), ((), ())),
                            preferred_element_type=jnp.float32)
        a = (g * jax.nn.sigmoid(g) * u).astype(jnp.bfloat16)
        o = lax.dot_general(a, w2_ref[0], (((1,), (1,)), ((), ())),
                            preferred_element_type=jnp.float32)
        o_ref[...] = o * ws_ref[:, 0:1]


def _make_dispatch(s, tpad):
    nsub = (2 * s // NW) // SUB
    mesh = plsc.VectorSubcoreMesh(core_axis_name="c", subcore_axis_name="s")

    @functools.partial(
        pl.kernel,
        out_type=(
            jax.ShapeDtypeStruct((tpad, 4, 128), jnp.int32),
            jax.ShapeDtypeStruct((tpad, 128), jnp.float32),
        ),
        mesh=mesh,
        scratch_types=[
            pltpu.VMEM((nsub, SUB), jnp.int32),
            pltpu.VMEM((SUB, 4, 128), jnp.int32),
            pltpu.VMEM((SUB, 128), jnp.float32),
            pltpu.SemaphoreType.DMA,
        ],
    )
    def dispatch(x_hbm, pos_hbm, wrep_hbm, xs_hbm, ws_hbm,
                 idx_v, rows_v, wrow_v, sem):
        wid = lax.axis_index("s") * 2 + lax.axis_index("c")
        base = wid * (nsub * SUB)
        tok_base = lax.rem(base, s)
        pltpu.sync_copy(pos_hbm.at[wid], idx_v)
        for i in range(nsub):
            pltpu.sync_copy(x_hbm.at[pl.ds(tok_base + i * SUB, SUB)], rows_v)
            pltpu.async_copy(rows_v, xs_hbm.at[idx_v.at[i]], sem).wait()
            pltpu.sync_copy(wrep_hbm.at[pl.ds(base + i * SUB, SUB)], wrow_v)
            pltpu.async_copy(wrow_v, ws_hbm.at[idx_v.at[i]], sem).wait()

    return dispatch


def _make_combine(s, dim, tpad):
    tpt = s // NW
    nh = tpt // HALF
    nc = dim // 16
    mesh = plsc.VectorSubcoreMesh(core_axis_name="c", subcore_axis_name="s")

    @functools.partial(
        pl.kernel,
        out_type=jax.ShapeDtypeStruct((s, dim), jnp.float32),
        mesh=mesh,
        scratch_types=[
            pltpu.VMEM((2 * nh, HALF), jnp.int32),
            pltpu.VMEM((HALF, dim), jnp.float32),
            pltpu.VMEM((HALF, dim), jnp.float32),
            pltpu.SemaphoreType.DMA,
        ],
    )
    def combine(out_hbm, pos_hbm, y_hbm, idx_v, buf_a, buf_b, sem):
        wid = lax.axis_index("s") * 2 + lax.axis_index("c")
        tbase = wid * tpt
        pltpu.sync_copy(pos_hbm.at[wid], idx_v)
        for h in range(nh):
            pltpu.async_copy(out_hbm.at[idx_v.at[h]], buf_a, sem).wait()
            pltpu.async_copy(out_hbm.at[idx_v.at[nh + h]], buf_b, sem).wait()

            def row_body(r, _):
                def col_body(cc, _):
                    a = buf_a[r, pl.ds(cc * 16, 16)]
                    b = buf_b[r, pl.ds(cc * 16, 16)]
                    buf_a[r, pl.ds(cc * 16, 16)] = a + b
                    return 0

                lax.fori_loop(0, nc, col_body, 0, unroll=8)
                return 0

            lax.fori_loop(0, HALF, row_body, 0)
            pltpu.sync_copy(buf_a, y_hbm.at[pl.ds(tbase + h * HALF, HALF)])

    return combine


def kernel(x, Wg, W1, W3, W2):
    orig_shape = x.shape
    dim = orig_shape[-1]
    xf = x.reshape(-1, dim)
    s = xf.shape[0]
    nexp, hid, _ = W1.shape
    tpad = ((2 * s + NEXP * (BT - 1) + BT - 1) // BT) * BT
    nb = tpad // BT

    pos, w, be, act = pl.pallas_call(
        _route_kernel,
        out_shape=[
            jax.ShapeDtypeStruct((1, 2 * s), jnp.int32),
            jax.ShapeDtypeStruct((1, 2 * s), jnp.float32),
            jax.ShapeDtypeStruct((1, NBP), jnp.int32),
            jax.ShapeDtypeStruct((1, NBP), jnp.int32),
        ],
    )(xf, Wg)

    xb = xf.astype(jnp.bfloat16)
    xi = lax.bitcast_convert_type(
        xb.reshape(s, dim // 256, 128, 2), jnp.int32)
    nsub = (2 * s // NW) // SUB
    wrep = jnp.broadcast_to(w.reshape(2 * s, 1), (2 * s, 128))
    xs3, ws = _make_dispatch(s, tpad)(xi, pos.reshape(NW, nsub, SUB), wrep)
    xs = lax.bitcast_convert_type(xs3, jnp.bfloat16).reshape(tpad, dim)

    w1b = W1.astype(jnp.bfloat16)
    w3b = W3.astype(jnp.bfloat16)
    w2b = W2.astype(jnp.bfloat16)

    outs = pl.pallas_call(
        _ffn_kernel,
        grid_spec=pltpu.PrefetchScalarGridSpec(
            num_scalar_prefetch=2,
            grid=(nb,),
            in_specs=[
                pl.BlockSpec((BT, dim), lambda b, be, act: (b, 0)),
                pl.BlockSpec((1, hid, dim), lambda b, be, act: (be[b], 0, 0)),
                pl.BlockSpec((1, hid, dim), lambda b, be, act: (be[b], 0, 0)),
                pl.BlockSpec((1, dim, hid), lambda b, be, act: (be[b], 0, 0)),
                pl.BlockSpec((BT, 128), lambda b, be, act: (b, 0)),
            ],
            out_specs=pl.BlockSpec((BT, dim), lambda b, be, act: (b, 0)),
        ),
        out_shape=jax.ShapeDtypeStruct((tpad, dim), jnp.float32),
    )(be.reshape(NBP), act.reshape(NBP), xs, w1b, w3b, w2b, ws)

    tpt = s // NW
    nh = tpt // HALF
    p0 = pos[0, :s].reshape(NW, nh, HALF)
    p1 = pos[0, s:].reshape(NW, nh, HALF)
    pos_t = jnp.concatenate([p0, p1], axis=1)

    y = _make_combine(s, dim, tpad)(outs, pos_t)
    return y.reshape(orig_shape)


# speedup vs baseline: 1.5919x; 1.5919x over previous
import functools

import jax
import jax.numpy as jnp
from jax import lax
from jax.experimental import pallas as pl
from jax.experimental.pallas import tpu as pltpu
from jax.experimental.pallas import tpu_sc as plsc

NEXP = 8
BT = 256
NBP = 32
NW = 32
SUB = 32
HALF = 32


def _route_kernel(x_ref, wg_ref, pos_ref, w_ref, be_ref, act_ref):
    scoresT = lax.dot_general(wg_ref[...], x_ref[...],
                              (((1,), (1,)), ((), ())),
                              preferred_element_type=jnp.float32)
    e_, s_ = scoresT.shape
    sub = lax.broadcasted_iota(jnp.int32, (e_, s_), 0)
    v1 = jnp.max(scoresT, axis=0, keepdims=True)
    i1 = jnp.min(jnp.where(scoresT == v1, sub, e_), axis=0, keepdims=True)
    masked = jnp.where(sub == i1, -jnp.inf, scoresT)
    v2 = jnp.max(masked, axis=0, keepdims=True)
    i2 = jnp.min(jnp.where(masked == v2, sub, e_), axis=0, keepdims=True)
    w1 = 1.0 / (1.0 + jnp.exp(v2 - v1))
    w_ref[...] = jnp.concatenate([w1, 1.0 - w1], axis=1)

    oh = jnp.concatenate([sub == i1, sub == i2], axis=1).astype(jnp.int32)
    c = oh
    sh = 1
    while sh < 2 * s_:
        c = c + jnp.concatenate(
            [jnp.zeros((e_, sh), jnp.int32), c[:, :-sh]], axis=1)
        sh *= 2
    excl = c - oh
    counts = c[:, -1:]
    pc = ((counts + BT - 1) // BT) * BT
    oc = pc
    sh = 1
    while sh < e_:
        oc = oc + jnp.concatenate(
            [jnp.zeros((sh, 1), jnp.int32), oc[:-sh]], axis=0)
        sh *= 2
    off = oc - pc
    end = oc
    pos_ref[...] = jnp.sum(oh * (off + excl), axis=0, keepdims=True)

    starts = lax.broadcasted_iota(jnp.int32, (1, NBP), 1) * BT
    nbefore = jnp.sum((starts >= end).astype(jnp.int32), axis=0, keepdims=True)
    be_ref[...] = jnp.minimum(nbefore, e_ - 1)
    act_ref[...] = (starts < end[e_ - 1:e_, :]).astype(jnp.int32)


def _ffn_kernel(be_ref, act_ref, xs_ref, w1_ref, w3_ref, w2_ref, ws_ref, o_ref):
    b = pl.program_id(0)

    @pl.when(act_ref[b] == 1)
    def _():
        xb = xs_ref[...].astype(jnp.bfloat16)
        g = lax.dot_general(xb, w1_ref[0], (((1,), (1,)), ((), ())),
                            preferred_element_type=jnp.float32)
        u = lax.dot_general(xb, w3_ref[0], (((1,), (1,)), ((), ())),
                            preferred_element_type=jnp.float32)
        a = (g * jax.nn.sigmoid(g) * u).astype(jnp.bfloat16)
        o = lax.dot_general(a, w2_ref[0], (((1,), (1,)), ((), ())),
                            preferred_element_type=jnp.float32)
        o_ref[...] = o * ws_ref[:, 0:1]


def _make_dispatch(s, dim, tpad):
    nsub = (2 * s // NW) // SUB
    mesh = plsc.VectorSubcoreMesh(core_axis_name="c", subcore_axis_name="s")

    @functools.partial(
        pl.kernel,
        out_type=(
            jax.ShapeDtypeStruct((tpad, dim), jnp.float32),
            jax.ShapeDtypeStruct((tpad, 128), jnp.float32),
        ),
        mesh=mesh,
        scratch_types=[
            pltpu.VMEM((nsub, SUB), jnp.int32),
            pltpu.VMEM((SUB, dim), jnp.float32),
            pltpu.VMEM((SUB, 128), jnp.float32),
            pltpu.SemaphoreType.DMA,
            pltpu.SemaphoreType.DMA,
        ],
    )
    def dispatch(x_hbm, pos_hbm, wrep_hbm, xs_hbm, ws_hbm,
                 idx_v, rows_v, wrow_v, sem, sem2):
        wid = lax.axis_index("s") * 2 + lax.axis_index("c")
        base = wid * (nsub * SUB)
        tok_base = lax.rem(base, s)
        pltpu.sync_copy(pos_hbm.at[wid], idx_v)
        for i in range(nsub):
            pltpu.sync_copy(x_hbm.at[pl.ds(tok_base + i * SUB, SUB)], rows_v)
            pltpu.sync_copy(wrep_hbm.at[pl.ds(base + i * SUB, SUB)], wrow_v)
            ca = pltpu.async_copy(rows_v, xs_hbm.at[idx_v.at[i]], sem)
            cb = pltpu.async_copy(wrow_v, ws_hbm.at[idx_v.at[i]], sem2)
            ca.wait()
            cb.wait()

    return dispatch


def _make_combine(s, dim, tpad):
    tpt = s // NW
    nh = tpt // HALF
    nc = dim // 16
    mesh = plsc.VectorSubcoreMesh(core_axis_name="c", subcore_axis_name="s")

    @functools.partial(
        pl.kernel,
        out_type=jax.ShapeDtypeStruct((s, dim), jnp.float32),
        mesh=mesh,
        scratch_types=[
            pltpu.VMEM((2 * nh, HALF), jnp.int32),
            pltpu.VMEM((HALF, dim), jnp.float32),
            pltpu.VMEM((HALF, dim), jnp.float32),
            pltpu.SemaphoreType.DMA,
            pltpu.SemaphoreType.DMA,
        ],
    )
    def combine(out_hbm, pos_hbm, y_hbm, idx_v, buf_a, buf_b, sem, sem2):
        wid = lax.axis_index("s") * 2 + lax.axis_index("c")
        tbase = wid * tpt
        pltpu.sync_copy(pos_hbm.at[wid], idx_v)
        for h in range(nh):
            ca = pltpu.async_copy(out_hbm.at[idx_v.at[h]], buf_a, sem)
            cb = pltpu.async_copy(out_hbm.at[idx_v.at[nh + h]], buf_b, sem2)
            ca.wait()
            cb.wait()

            def row_body(r, _):
                def col_body(cc, _):
                    a = buf_a[r, pl.ds(cc * 16, 16)]
                    b = buf_b[r, pl.ds(cc * 16, 16)]
                    buf_a[r, pl.ds(cc * 16, 16)] = a + b
                    return 0

                lax.fori_loop(0, nc, col_body, 0, unroll=8)
                return 0

            lax.fori_loop(0, HALF, row_body, 0)
            pltpu.sync_copy(buf_a, y_hbm.at[pl.ds(tbase + h * HALF, HALF)])

    return combine


def kernel(x, Wg, W1, W3, W2):
    orig_shape = x.shape
    dim = orig_shape[-1]
    xf = x.reshape(-1, dim)
    s = xf.shape[0]
    nexp, hid, _ = W1.shape
    tpad = ((2 * s + NEXP * (BT - 1) + BT - 1) // BT) * BT
    nb = tpad // BT

    pos, w, be, act = pl.pallas_call(
        _route_kernel,
        out_shape=[
            jax.ShapeDtypeStruct((1, 2 * s), jnp.int32),
            jax.ShapeDtypeStruct((1, 2 * s), jnp.float32),
            jax.ShapeDtypeStruct((1, NBP), jnp.int32),
            jax.ShapeDtypeStruct((1, NBP), jnp.int32),
        ],
    )(xf, Wg)

    nsub = (2 * s // NW) // SUB
    wrep = jnp.broadcast_to(w.reshape(2 * s, 1), (2 * s, 128))
    xs, ws = _make_dispatch(s, dim, tpad)(
        xf, pos.reshape(NW, nsub, SUB), wrep)

    w1b = W1.astype(jnp.bfloat16)
    w3b = W3.astype(jnp.bfloat16)
    w2b = W2.astype(jnp.bfloat16)

    outs = pl.pallas_call(
        _ffn_kernel,
        grid_spec=pltpu.PrefetchScalarGridSpec(
            num_scalar_prefetch=2,
            grid=(nb,),
            in_specs=[
                pl.BlockSpec((BT, dim), lambda b, be, act: (b, 0)),
                pl.BlockSpec((1, hid, dim), lambda b, be, act: (be[b], 0, 0)),
                pl.BlockSpec((1, hid, dim), lambda b, be, act: (be[b], 0, 0)),
                pl.BlockSpec((1, dim, hid), lambda b, be, act: (be[b], 0, 0)),
                pl.BlockSpec((BT, 128), lambda b, be, act: (b, 0)),
            ],
            out_specs=pl.BlockSpec((BT, dim), lambda b, be, act: (b, 0)),
        ),
        out_shape=jax.ShapeDtypeStruct((tpad, dim), jnp.float32),
    )(be.reshape(NBP), act.reshape(NBP), xs, w1b, w3b, w2b, ws)

    tpt = s // NW
    nh = tpt // HALF
    p0 = pos[0, :s].reshape(NW, nh, HALF)
    p1 = pos[0, s:].reshape(NW, nh, HALF)
    pos_t = jnp.concatenate([p0, p1], axis=1)

    y = _make_combine(s, dim, tpad)(outs, pos_t)
    return y.reshape(orig_shape)


# speedup vs baseline: 1.8459x; 1.1595x over previous
import functools

import jax
import jax.numpy as jnp
from jax import lax
from jax.experimental import pallas as pl
from jax.experimental.pallas import tpu as pltpu
from jax.experimental.pallas import tpu_sc as plsc

NEXP = 8
BT = 256
NBP = 32
NW = 32
SUB = 32
HALF = 32


def _route_kernel(x_ref, wg_ref, pos_ref, w_ref, be_ref, act_ref):
    scoresT = lax.dot_general(wg_ref[...], x_ref[...],
                              (((1,), (1,)), ((), ())),
                              preferred_element_type=jnp.float32)
    e_, s_ = scoresT.shape
    sub = lax.broadcasted_iota(jnp.int32, (e_, s_), 0)
    v1 = jnp.max(scoresT, axis=0, keepdims=True)
    i1 = jnp.min(jnp.where(scoresT == v1, sub, e_), axis=0, keepdims=True)
    masked = jnp.where(sub == i1, -jnp.inf, scoresT)
    v2 = jnp.max(masked, axis=0, keepdims=True)
    i2 = jnp.min(jnp.where(masked == v2, sub, e_), axis=0, keepdims=True)
    w1 = 1.0 / (1.0 + jnp.exp(v2 - v1))
    w_ref[...] = jnp.concatenate([w1, 1.0 - w1], axis=1)

    oh = jnp.concatenate([sub == i1, sub == i2], axis=1).astype(jnp.int32)
    c = oh
    sh = 1
    while sh < 2 * s_:
        c = c + jnp.concatenate(
            [jnp.zeros((e_, sh), jnp.int32), c[:, :-sh]], axis=1)
        sh *= 2
    excl = c - oh
    counts = c[:, -1:]
    pc = ((counts + BT - 1) // BT) * BT
    oc = pc
    sh = 1
    while sh < e_:
        oc = oc + jnp.concatenate(
            [jnp.zeros((sh, 1), jnp.int32), oc[:-sh]], axis=0)
        sh *= 2
    off = oc - pc
    end = oc
    pos_ref[...] = jnp.sum(oh * (off + excl), axis=0, keepdims=True)

    starts = lax.broadcasted_iota(jnp.int32, (1, NBP), 1) * BT
    nbefore = jnp.sum((starts >= end).astype(jnp.int32), axis=0, keepdims=True)
    be_ref[...] = jnp.minimum(nbefore, e_ - 1)
    act_ref[...] = (starts < end[e_ - 1:e_, :]).astype(jnp.int32)


def _ffn_part(xs_ref, w1_ref, w3_ref, w2_ref, ws_ref):
    xb = xs_ref[...].astype(jnp.bfloat16)
    g = lax.dot_general(xb, w1_ref[0], (((1,), (1,)), ((), ())),
                        preferred_element_type=jnp.float32)
    u = lax.dot_general(xb, w3_ref[0], (((1,), (1,)), ((), ())),
                        preferred_element_type=jnp.float32)
    a = (g * jax.nn.sigmoid(g) * u).astype(jnp.bfloat16)
    o = lax.dot_general(a, w2_ref[0], (((1,), (1,)), ((), ())),
                        preferred_element_type=jnp.float32)
    return o * ws_ref[:, 0:1]


def _ffn_kernel(be_ref, act_ref, xs_ref, w1_ref, w3_ref, w2_ref, ws_ref, o_ref):
    b = pl.program_id(0)

    @pl.when(act_ref[b] == 1)
    def _():
        o_ref[...] = _ffn_part(xs_ref, w1_ref, w3_ref, w2_ref, ws_ref)


def _ffn2_kernel(be_ref, act_ref, xs_ref, w1_ref, w3_ref, w2_ref, ws_ref,
                 prev_ref, o_ref):
    b = pl.program_id(0)

    @pl.when(act_ref[b] == 1)
    def _():
        o_ref[...] = prev_ref[...] + _ffn_part(
            xs_ref, w1_ref, w3_ref, w2_ref, ws_ref)


def _make_dispatch(s, dim, tpad):
    nsub = (2 * s // NW) // SUB
    mesh = plsc.VectorSubcoreMesh(core_axis_name="c", subcore_axis_name="s")

    @functools.partial(
        pl.kernel,
        out_type=(
            jax.ShapeDtypeStruct((tpad, dim), jnp.float32),
            jax.ShapeDtypeStruct((tpad, 128), jnp.float32),
        ),
        mesh=mesh,
        scratch_types=[
            pltpu.VMEM((nsub, SUB), jnp.int32),
            pltpu.VMEM((SUB, dim), jnp.float32),
            pltpu.VMEM((SUB, 128), jnp.float32),
            pltpu.SemaphoreType.DMA,
            pltpu.SemaphoreType.DMA,
        ],
    )
    def dispatch(x_hbm, pos_hbm, wrep_hbm, xs_hbm, ws_hbm,
                 idx_v, rows_v, wrow_v, sem, sem2):
        wid = lax.axis_index("s") * 2 + lax.axis_index("c")
        base = wid * (nsub * SUB)
        tok_base = lax.rem(base, s)
        pltpu.sync_copy(pos_hbm.at[wid], idx_v)
        for i in range(nsub):
            pltpu.sync_copy(x_hbm.at[pl.ds(tok_base + i * SUB, SUB)], rows_v)
            pltpu.sync_copy(wrep_hbm.at[pl.ds(base + i * SUB, SUB)], wrow_v)
            ca = pltpu.async_copy(rows_v, xs_hbm.at[idx_v.at[i]], sem)
            cb = pltpu.async_copy(wrow_v, ws_hbm.at[idx_v.at[i]], sem2)
            ca.wait()
            cb.wait()

    return dispatch


def _make_combine(s, dim, tpad):
    tpt = s // NW
    nh = tpt // HALF
    nc = dim // 16
    mesh = plsc.VectorSubcoreMesh(core_axis_name="c", subcore_axis_name="s")

    @functools.partial(
        pl.kernel,
        out_type=jax.ShapeDtypeStruct((s, dim), jnp.float32),
        mesh=mesh,
        scratch_types=[
            pltpu.VMEM((2 * nh, HALF), jnp.int32),
            pltpu.VMEM((HALF, dim), jnp.float32),
            pltpu.VMEM((HALF, dim), jnp.float32),
            pltpu.SemaphoreType.DMA,
            pltpu.SemaphoreType.DMA,
        ],
    )
    def combine(out_hbm, pos_hbm, y_hbm, idx_v, buf_a, buf_b, sem, sem2):
        wid = lax.axis_index("s") * 2 + lax.axis_index("c")
        tbase = wid * tpt
        pltpu.sync_copy(pos_hbm.at[wid], idx_v)
        for h in range(nh):
            ca = pltpu.async_copy(out_hbm.at[idx_v.at[h]], buf_a, sem)
            cb = pltpu.async_copy(out_hbm.at[idx_v.at[nh + h]], buf_b, sem2)
            ca.wait()
            cb.wait()

            def row_body(r, _):
                def col_body(cc, _):
                    a = buf_a[r, pl.ds(cc * 16, 16)]
                    b = buf_b[r, pl.ds(cc * 16, 16)]
                    buf_a[r, pl.ds(cc * 16, 16)] = a + b
                    return 0

                lax.fori_loop(0, nc, col_body, 0, unroll=8)
                return 0

            lax.fori_loop(0, HALF, row_body, 0)
            pltpu.sync_copy(buf_a, y_hbm.at[pl.ds(tbase + h * HALF, HALF)])

    return combine


def kernel(x, Wg, W1, W3, W2):
    orig_shape = x.shape
    dim = orig_shape[-1]
    xf = x.reshape(-1, dim)
    s = xf.shape[0]
    nexp, hid, _ = W1.shape
    tpad = ((2 * s + NEXP * (BT - 1) + BT - 1) // BT) * BT
    nb = tpad // BT

    pos, w, be, act = pl.pallas_call(
        _route_kernel,
        out_shape=[
            jax.ShapeDtypeStruct((1, 2 * s), jnp.int32),
            jax.ShapeDtypeStruct((1, 2 * s), jnp.float32),
            jax.ShapeDtypeStruct((1, NBP), jnp.int32),
            jax.ShapeDtypeStruct((1, NBP), jnp.int32),
        ],
    )(xf, Wg)

    nsub = (2 * s // NW) // SUB
    wrep = jnp.broadcast_to(w.reshape(2 * s, 1), (2 * s, 128))
    xs, ws = _make_dispatch(s, dim, tpad)(
        xf, pos.reshape(NW, nsub, SUB), wrep)

    hid2 = hid // 2
    common_specs = [
        pl.BlockSpec((BT, dim), lambda b, be, act: (b, 0)),
    ]

    def wspecs(hh):
        return [
            pl.BlockSpec((1, hid2, dim), lambda b, be, act: (be[b], hh, 0)),
            pl.BlockSpec((1, hid2, dim), lambda b, be, act: (be[b], hh, 0)),
            pl.BlockSpec((1, dim, hid2), lambda b, be, act: (be[b], 0, hh)),
            pl.BlockSpec((BT, 128), lambda b, be, act: (b, 0)),
        ]

    outs0 = pl.pallas_call(
        _ffn_kernel,
        grid_spec=pltpu.PrefetchScalarGridSpec(
            num_scalar_prefetch=2,
            grid=(nb,),
            in_specs=common_specs + wspecs(0),
            out_specs=pl.BlockSpec((BT, dim), lambda b, be, act: (b, 0)),
        ),
        out_shape=jax.ShapeDtypeStruct((tpad, dim), jnp.float32),
    )(be.reshape(NBP), act.reshape(NBP), xs, W1, W3, W2, ws)

    outs = pl.pallas_call(
        _ffn2_kernel,
        grid_spec=pltpu.PrefetchScalarGridSpec(
            num_scalar_prefetch=2,
            grid=(nb,),
            in_specs=common_specs + wspecs(1)
            + [pl.BlockSpec((BT, dim), lambda b, be, act: (b, 0))],
            out_specs=pl.BlockSpec((BT, dim), lambda b, be, act: (b, 0)),
        ),
        out_shape=jax.ShapeDtypeStruct((tpad, dim), jnp.float32),
        input_output_aliases={7: 0},
    )(be.reshape(NBP), act.reshape(NBP), xs, W1, W3, W2, ws, outs0)

    tpt = s // NW
    nh = tpt // HALF
    p0 = pos[0, :s].reshape(NW, nh, HALF)
    p1 = pos[0, s:].reshape(NW, nh, HALF)
    pos_t = jnp.concatenate([p0, p1], axis=1)

    y = _make_combine(s, dim, tpad)(outs, pos_t)
    return y.reshape(orig_shape)


# speedup vs baseline: 1.9968x; 1.0818x over previous
import functools

import jax
import jax.numpy as jnp
from jax import lax
from jax.experimental import pallas as pl
from jax.experimental.pallas import tpu as pltpu
from jax.experimental.pallas import tpu_sc as plsc

NEXP = 8
BT = 512
NBP = 32
NW = 32
SUB = 32
HALF = 32


def _route_kernel(x_ref, wg_ref, pos_ref, w_ref, be_ref, act_ref):
    scoresT = lax.dot_general(wg_ref[...], x_ref[...],
                              (((1,), (1,)), ((), ())),
                              preferred_element_type=jnp.float32)
    e_, s_ = scoresT.shape
    sub = lax.broadcasted_iota(jnp.int32, (e_, s_), 0)
    v1 = jnp.max(scoresT, axis=0, keepdims=True)
    i1 = jnp.min(jnp.where(scoresT == v1, sub, e_), axis=0, keepdims=True)
    masked = jnp.where(sub == i1, -jnp.inf, scoresT)
    v2 = jnp.max(masked, axis=0, keepdims=True)
    i2 = jnp.min(jnp.where(masked == v2, sub, e_), axis=0, keepdims=True)
    w1 = 1.0 / (1.0 + jnp.exp(v2 - v1))
    w_ref[...] = jnp.concatenate([w1, 1.0 - w1], axis=1)

    oh = jnp.concatenate([sub == i1, sub == i2], axis=1).astype(jnp.int32)
    c = oh
    sh = 1
    while sh < 2 * s_:
        c = c + jnp.concatenate(
            [jnp.zeros((e_, sh), jnp.int32), c[:, :-sh]], axis=1)
        sh *= 2
    excl = c - oh
    counts = c[:, -1:]
    pc = ((counts + BT - 1) // BT) * BT
    oc = pc
    sh = 1
    while sh < e_:
        oc = oc + jnp.concatenate(
            [jnp.zeros((sh, 1), jnp.int32), oc[:-sh]], axis=0)
        sh *= 2
    off = oc - pc
    end = oc
    pos_ref[...] = jnp.sum(oh * (off + excl), axis=0, keepdims=True)

    starts = lax.broadcasted_iota(jnp.int32, (1, NBP), 1) * BT
    nbefore = jnp.sum((starts >= end).astype(jnp.int32), axis=0, keepdims=True)
    be_ref[...] = jnp.minimum(nbefore, e_ - 1)
    act_ref[...] = (starts < end[e_ - 1:e_, :]).astype(jnp.int32)


def _ffn_part(xs_ref, w1_ref, w3_ref, w2_ref, ws_ref):
    xb = xs_ref[...].astype(jnp.bfloat16)
    g = lax.dot_general(xb, w1_ref[0], (((1,), (1,)), ((), ())),
                        preferred_element_type=jnp.float32)
    u = lax.dot_general(xb, w3_ref[0], (((1,), (1,)), ((), ())),
                        preferred_element_type=jnp.float32)
    a = (g * jax.nn.sigmoid(g) * u).astype(jnp.bfloat16)
    o = lax.dot_general(a, w2_ref[0], (((1,), (1,)), ((), ())),
                        preferred_element_type=jnp.float32)
    return o * ws_ref[:, 0:1]


def _ffn_kernel(be_ref, act_ref, xs_ref, w1_ref, w3_ref, w2_ref, ws_ref, o_ref):
    b = pl.program_id(0)

    @pl.when(act_ref[b] == 1)
    def _():
        o_ref[...] = _ffn_part(xs_ref, w1_ref, w3_ref, w2_ref, ws_ref)


def _ffn2_kernel(be_ref, act_ref, xs_ref, w1_ref, w3_ref, w2_ref, ws_ref,
                 prev_ref, o_ref):
    b = pl.program_id(0)

    @pl.when(act_ref[b] == 1)
    def _():
        o_ref[...] = prev_ref[...] + _ffn_part(
            xs_ref, w1_ref, w3_ref, w2_ref, ws_ref)


def _make_dispatch(s, dim, tpad):
    nsub = (2 * s // NW) // SUB
    mesh = plsc.VectorSubcoreMesh(core_axis_name="c", subcore_axis_name="s")

    @functools.partial(
        pl.kernel,
        out_type=(
            jax.ShapeDtypeStruct((tpad, dim), jnp.float32),
            jax.ShapeDtypeStruct((tpad, 128), jnp.float32),
        ),
        mesh=mesh,
        scratch_types=[
            pltpu.VMEM((nsub, SUB), jnp.int32),
            pltpu.VMEM((SUB, dim), jnp.float32),
            pltpu.VMEM((SUB, 128), jnp.float32),
            pltpu.SemaphoreType.DMA,
            pltpu.SemaphoreType.DMA,
        ],
    )
    def dispatch(x_hbm, pos_hbm, wrep_hbm, xs_hbm, ws_hbm,
                 idx_v, rows_v, wrow_v, sem, sem2):
        wid = lax.axis_index("s") * 2 + lax.axis_index("c")
        base = wid * (nsub * SUB)
        tok_base = lax.rem(base, s)
        pltpu.sync_copy(pos_hbm.at[wid], idx_v)
        for i in range(nsub):
            pltpu.sync_copy(x_hbm.at[pl.ds(tok_base + i * SUB, SUB)], rows_v)
            pltpu.sync_copy(wrep_hbm.at[pl.ds(base + i * SUB, SUB)], wrow_v)
            ca = pltpu.async_copy(rows_v, xs_hbm.at[idx_v.at[i]], sem)
            cb = pltpu.async_copy(wrow_v, ws_hbm.at[idx_v.at[i]], sem2)
            ca.wait()
            cb.wait()

    return dispatch


def _make_combine(s, dim, tpad):
    tpt = s // NW
    nh = tpt // HALF
    nc = dim // 16
    mesh = plsc.VectorSubcoreMesh(core_axis_name="c", subcore_axis_name="s")

    @functools.partial(
        pl.kernel,
        out_type=jax.ShapeDtypeStruct((s, dim), jnp.float32),
        mesh=mesh,
        scratch_types=[
            pltpu.VMEM((2 * nh, HALF), jnp.int32),
            pltpu.VMEM((HALF, dim), jnp.float32),
            pltpu.VMEM((HALF, dim), jnp.float32),
            pltpu.SemaphoreType.DMA,
            pltpu.SemaphoreType.DMA,
        ],
    )
    def combine(out_hbm, pos_hbm, y_hbm, idx_v, buf_a, buf_b, sem, sem2):
        wid = lax.axis_index("s") * 2 + lax.axis_index("c")
        tbase = wid * tpt
        pltpu.sync_copy(pos_hbm.at[wid], idx_v)
        for h in range(nh):
            ca = pltpu.async_copy(out_hbm.at[idx_v.at[h]], buf_a, sem)
            cb = pltpu.async_copy(out_hbm.at[idx_v.at[nh + h]], buf_b, sem2)
            ca.wait()
            cb.wait()

            def row_body(r, _):
                def col_body(cc, _):
                    a = buf_a[r, pl.ds(cc * 16, 16)]
                    b = buf_b[r, pl.ds(cc * 16, 16)]
                    buf_a[r, pl.ds(cc * 16, 16)] = a + b
                    return 0

                lax.fori_loop(0, nc, col_body, 0, unroll=8)
                return 0

            lax.fori_loop(0, HALF, row_body, 0)
            pltpu.sync_copy(buf_a, y_hbm.at[pl.ds(tbase + h * HALF, HALF)])

    return combine


def kernel(x, Wg, W1, W3, W2):
    orig_shape = x.shape
    dim = orig_shape[-1]
    xf = x.reshape(-1, dim)
    s = xf.shape[0]
    nexp, hid, _ = W1.shape
    tpad = ((2 * s + NEXP * (BT - 1) + BT - 1) // BT) * BT
    nb = tpad // BT

    pos, w, be, act = pl.pallas_call(
        _route_kernel,
        out_shape=[
            jax.ShapeDtypeStruct((1, 2 * s), jnp.int32),
            jax.ShapeDtypeStruct((1, 2 * s), jnp.float32),
            jax.ShapeDtypeStruct((1, NBP), jnp.int32),
            jax.ShapeDtypeStruct((1, NBP), jnp.int32),
        ],
    )(xf, Wg)

    nsub = (2 * s // NW) // SUB
    wrep = jnp.broadcast_to(w.reshape(2 * s, 1), (2 * s, 128))
    xs, ws = _make_dispatch(s, dim, tpad)(
        xf, pos.reshape(NW, nsub, SUB), wrep)

    hid2 = hid // 2
    common_specs = [
        pl.BlockSpec((BT, dim), lambda b, be, act: (b, 0)),
    ]

    def wspecs(hh):
        return [
            pl.BlockSpec((1, hid2, dim), lambda b, be, act: (be[b], hh, 0)),
            pl.BlockSpec((1, hid2, dim), lambda b, be, act: (be[b], hh, 0)),
            pl.BlockSpec((1, dim, hid2), lambda b, be, act: (be[b], 0, hh)),
            pl.BlockSpec((BT, 128), lambda b, be, act: (b, 0)),
        ]

    outs0 = pl.pallas_call(
        _ffn_kernel,
        grid_spec=pltpu.PrefetchScalarGridSpec(
            num_scalar_prefetch=2,
            grid=(nb,),
            in_specs=common_specs + wspecs(0),
            out_specs=pl.BlockSpec((BT, dim), lambda b, be, act: (b, 0)),
        ),
        out_shape=jax.ShapeDtypeStruct((tpad, dim), jnp.float32),
    )(be.reshape(NBP), act.reshape(NBP), xs, W1, W3, W2, ws)

    outs = pl.pallas_call(
        _ffn2_kernel,
        grid_spec=pltpu.PrefetchScalarGridSpec(
            num_scalar_prefetch=2,
            grid=(nb,),
            in_specs=common_specs + wspecs(1)
            + [pl.BlockSpec((BT, dim), lambda b, be, act: (b, 0))],
            out_specs=pl.BlockSpec((BT, dim), lambda b, be, act: (b, 0)),
        ),
        out_shape=jax.ShapeDtypeStruct((tpad, dim), jnp.float32),
        input_output_aliases={7: 0},
    )(be.reshape(NBP), act.reshape(NBP), xs, W1, W3, W2, ws, outs0)

    tpt = s // NW
    nh = tpt // HALF
    p0 = pos[0, :s].reshape(NW, nh, HALF)
    p1 = pos[0, s:].reshape(NW, nh, HALF)
    pos_t = jnp.concatenate([p0, p1], axis=1)

    y = _make_combine(s, dim, tpad)(outs, pos_t)
    return y.reshape(orig_shape)


# speedup vs baseline: 2.0566x; 1.0299x over previous
import functools

import jax
import jax.numpy as jnp
from jax import lax
from jax.experimental import pallas as pl
from jax.experimental.pallas import tpu as pltpu
from jax.experimental.pallas import tpu_sc as plsc

NEXP = 8
BT = 512
NBP = 32
NW = 32
SUB = 32
HALF = 16


def _route_kernel(x_ref, wg_ref, pos_ref, w_ref, be_ref, act_ref):
    scoresT = lax.dot_general(wg_ref[...], x_ref[...],
                              (((1,), (1,)), ((), ())),
                              preferred_element_type=jnp.float32)
    e_, s_ = scoresT.shape
    sub = lax.broadcasted_iota(jnp.int32, (e_, s_), 0)
    v1 = jnp.max(scoresT, axis=0, keepdims=True)
    i1 = jnp.min(jnp.where(scoresT == v1, sub, e_), axis=0, keepdims=True)
    masked = jnp.where(sub == i1, -jnp.inf, scoresT)
    v2 = jnp.max(masked, axis=0, keepdims=True)
    i2 = jnp.min(jnp.where(masked == v2, sub, e_), axis=0, keepdims=True)
    w1 = 1.0 / (1.0 + jnp.exp(v2 - v1))
    w_ref[...] = jnp.concatenate([w1, 1.0 - w1], axis=1)

    oh = jnp.concatenate([sub == i1, sub == i2], axis=1).astype(jnp.int32)
    c = oh
    sh = 1
    while sh < 2 * s_:
        c = c + jnp.concatenate(
            [jnp.zeros((e_, sh), jnp.int32), c[:, :-sh]], axis=1)
        sh *= 2
    excl = c - oh
    counts = c[:, -1:]
    pc = ((counts + BT - 1) // BT) * BT
    oc = pc
    sh = 1
    while sh < e_:
        oc = oc + jnp.concatenate(
            [jnp.zeros((sh, 1), jnp.int32), oc[:-sh]], axis=0)
        sh *= 2
    off = oc - pc
    end = oc
    pos_ref[...] = jnp.sum(oh * (off + excl), axis=0, keepdims=True)

    starts = lax.broadcasted_iota(jnp.int32, (1, NBP), 1) * BT
    nbefore = jnp.sum((starts >= end).astype(jnp.int32), axis=0, keepdims=True)
    be_ref[...] = jnp.minimum(nbefore, e_ - 1)
    act_ref[...] = (starts < end[e_ - 1:e_, :]).astype(jnp.int32)


def _ffn_part(xs_ref, w1_ref, w3_ref, w2_ref, ws_ref):
    xb = xs_ref[...].astype(jnp.bfloat16)
    g = lax.dot_general(xb, w1_ref[0], (((1,), (1,)), ((), ())),
                        preferred_element_type=jnp.float32)
    u = lax.dot_general(xb, w3_ref[0], (((1,), (1,)), ((), ())),
                        preferred_element_type=jnp.float32)
    a = (g * jax.nn.sigmoid(g) * u).astype(jnp.bfloat16)
    o = lax.dot_general(a, w2_ref[0], (((1,), (1,)), ((), ())),
                        preferred_element_type=jnp.float32)
    return o * ws_ref[:, 0:1]


def _ffn_kernel(be_ref, act_ref, xs_ref, w1_ref, w3_ref, w2_ref, ws_ref, o_ref):
    b = pl.program_id(0)

    @pl.when(act_ref[b] == 1)
    def _():
        o_ref[...] = _ffn_part(xs_ref, w1_ref, w3_ref, w2_ref, ws_ref)


def _ffn2_kernel(be_ref, act_ref, xs_ref, w1_ref, w3_ref, w2_ref, ws_ref,
                 prev_ref, o_ref):
    b = pl.program_id(0)

    @pl.when(act_ref[b] == 1)
    def _():
        o_ref[...] = prev_ref[...] + _ffn_part(
            xs_ref, w1_ref, w3_ref, w2_ref, ws_ref)


def _make_dispatch(s, dim, tpad):
    nsub = (2 * s // NW) // SUB
    mesh = plsc.VectorSubcoreMesh(core_axis_name="c", subcore_axis_name="s")

    apw = nsub * SUB

    @functools.partial(
        pl.kernel,
        out_type=(
            jax.ShapeDtypeStruct((tpad, dim), jnp.float32),
            jax.ShapeDtypeStruct((tpad, 128), jnp.float32),
        ),
        mesh=mesh,
        scratch_types=[
            pltpu.VMEM((nsub, SUB), jnp.int32),
            pltpu.VMEM((apw,), jnp.int32),
            pltpu.VMEM((SUB, dim), jnp.float32),
            pltpu.VMEM((SUB, dim), jnp.float32),
            pltpu.VMEM((apw, 128), jnp.float32),
            pltpu.SemaphoreType.DMA,
            pltpu.SemaphoreType.DMA,
            pltpu.SemaphoreType.DMA,
            pltpu.SemaphoreType.DMA,
            pltpu.SemaphoreType.DMA,
        ],
    )
    def dispatch(x_hbm, pos_hbm, posf_hbm, wrep_hbm, xs_hbm, ws_hbm,
                 idx_v, idxf_v, r0, r1, wbig,
                 semr0, semr1, sems0, sems1, semw):
        wid = lax.axis_index("s") * 2 + lax.axis_index("c")
        base = wid * apw
        tok_base = lax.rem(base, s)
        pltpu.sync_copy(pos_hbm.at[wid], idx_v)
        pltpu.sync_copy(posf_hbm.at[wid], idxf_v)
        pltpu.sync_copy(wrep_hbm.at[pl.ds(base, apw)], wbig)
        cw = pltpu.async_copy(wbig, ws_hbm.at[idxf_v], semw)
        rbufs = (r0, r1)
        rsems = (semr0, semr1)
        ssems = (sems0, sems1)
        pend_r = [None, None]
        pend_s = [None, None]

        def startr(i):
            p = i % 2
            if pend_s[p] is not None:
                pend_s[p].wait()
                pend_s[p] = None
            pend_r[p] = pltpu.async_copy(
                x_hbm.at[pl.ds(tok_base + i * SUB, SUB)], rbufs[p], rsems[p])

        startr(0)
        for i in range(nsub):
            p = i % 2
            if i + 1 < nsub:
                startr(i + 1)
            pend_r[p].wait()
            pend_s[p] = pltpu.async_copy(
                rbufs[p], xs_hbm.at[idx_v.at[i]], ssems[p])
        for p in (0, 1):
            if pend_s[p] is not None:
                pend_s[p].wait()
        cw.wait()

    return dispatch


def _make_combine(s, dim, tpad):
    tpt = s // NW
    nh = tpt // HALF
    nc = dim // 16
    mesh = plsc.VectorSubcoreMesh(core_axis_name="c", subcore_axis_name="s")

    @functools.partial(
        pl.kernel,
        out_type=jax.ShapeDtypeStruct((s, dim), jnp.float32),
        mesh=mesh,
        scratch_types=[
            pltpu.VMEM((2 * nh, HALF), jnp.int32),
            pltpu.VMEM((HALF, dim), jnp.float32),
            pltpu.VMEM((HALF, dim), jnp.float32),
            pltpu.VMEM((HALF, dim), jnp.float32),
            pltpu.VMEM((HALF, dim), jnp.float32),
            pltpu.SemaphoreType.DMA,
            pltpu.SemaphoreType.DMA,
            pltpu.SemaphoreType.DMA,
            pltpu.SemaphoreType.DMA,
        ],
    )
    def combine(out_hbm, pos_hbm, y_hbm, idx_v,
                ba0, bb0, ba1, bb1, semg0, semg1, semw0, semw1):
        wid = lax.axis_index("s") * 2 + lax.axis_index("c")
        tbase = wid * tpt
        pltpu.sync_copy(pos_hbm.at[wid], idx_v)
        bufs_a = (ba0, ba1)
        bufs_b = (bb0, bb1)
        gsems = (semg0, semg1)
        wsems = (semw0, semw1)
        pend_g = [None, None]
        pend_w = [None, None]

        def startg(h):
            p = h % 2
            if pend_w[p] is not None:
                pend_w[p].wait()
                pend_w[p] = None
            pend_g[p] = (
                pltpu.async_copy(out_hbm.at[idx_v.at[h]], bufs_a[p], gsems[p]),
                pltpu.async_copy(out_hbm.at[idx_v.at[nh + h]], bufs_b[p],
                                 gsems[p]))

        startg(0)
        for h in range(nh):
            p = h % 2
            if h + 1 < nh:
                startg(h + 1)
            ca, cb = pend_g[p]
            ca.wait()
            cb.wait()
            buf_a = bufs_a[p]
            buf_b = bufs_b[p]

            def row_body(r, _):
                def col_body(cc, _):
                    a = buf_a[r, pl.ds(cc * 16, 16)]
                    b = buf_b[r, pl.ds(cc * 16, 16)]
                    buf_a[r, pl.ds(cc * 16, 16)] = a + b
                    return 0

                lax.fori_loop(0, nc, col_body, 0, unroll=8)
                return 0

            lax.fori_loop(0, HALF, row_body, 0)
            pend_w[p] = pltpu.async_copy(
                buf_a, y_hbm.at[pl.ds(tbase + h * HALF, HALF)], wsems[p])
        for p in (0, 1):
            if pend_w[p] is not None:
                pend_w[p].wait()

    return combine


def kernel(x, Wg, W1, W3, W2):
    orig_shape = x.shape
    dim = orig_shape[-1]
    xf = x.reshape(-1, dim)
    s = xf.shape[0]
    nexp, hid, _ = W1.shape
    tpad = ((2 * s + NEXP * (BT - 1) + BT - 1) // BT) * BT
    nb = tpad // BT

    pos, w, be, act = pl.pallas_call(
        _route_kernel,
        out_shape=[
            jax.ShapeDtypeStruct((1, 2 * s), jnp.int32),
            jax.ShapeDtypeStruct((1, 2 * s), jnp.float32),
            jax.ShapeDtypeStruct((1, NBP), jnp.int32),
            jax.ShapeDtypeStruct((1, NBP), jnp.int32),
        ],
    )(xf, Wg)

    nsub = (2 * s // NW) // SUB
    wrep = jnp.broadcast_to(w.reshape(2 * s, 1), (2 * s, 128))
    xs, ws = _make_dispatch(s, dim, tpad)(
        xf, pos.reshape(NW, nsub, SUB), pos.reshape(NW, nsub * SUB), wrep)

    hid2 = hid // 2
    common_specs = [
        pl.BlockSpec((BT, dim), lambda b, be, act: (b, 0)),
    ]

    def wspecs(hh):
        return [
            pl.BlockSpec((1, hid2, dim), lambda b, be, act: (be[b], hh, 0)),
            pl.BlockSpec((1, hid2, dim), lambda b, be, act: (be[b], hh, 0)),
            pl.BlockSpec((1, dim, hid2), lambda b, be, act: (be[b], 0, hh)),
            pl.BlockSpec((BT, 128), lambda b, be, act: (b, 0)),
        ]

    outs0 = pl.pallas_call(
        _ffn_kernel,
        grid_spec=pltpu.PrefetchScalarGridSpec(
            num_scalar_prefetch=2,
            grid=(nb,),
            in_specs=common_specs + wspecs(0),
            out_specs=pl.BlockSpec((BT, dim), lambda b, be, act: (b, 0)),
        ),
        out_shape=jax.ShapeDtypeStruct((tpad, dim), jnp.float32),
    )(be.reshape(NBP), act.reshape(NBP), xs, W1, W3, W2, ws)

    outs = pl.pallas_call(
        _ffn2_kernel,
        grid_spec=pltpu.PrefetchScalarGridSpec(
            num_scalar_prefetch=2,
            grid=(nb,),
            in_specs=common_specs + wspecs(1)
            + [pl.BlockSpec((BT, dim), lambda b, be, act: (b, 0))],
            out_specs=pl.BlockSpec((BT, dim), lambda b, be, act: (b, 0)),
        ),
        out_shape=jax.ShapeDtypeStruct((tpad, dim), jnp.float32),
        input_output_aliases={7: 0},
    )(be.reshape(NBP), act.reshape(NBP), xs, W1, W3, W2, ws, outs0)

    tpt = s // NW
    nh = tpt // HALF
    p0 = pos[0, :s].reshape(NW, nh, HALF)
    p1 = pos[0, s:].reshape(NW, nh, HALF)
    pos_t = jnp.concatenate([p0, p1], axis=1)

    y = _make_combine(s, dim, tpad)(outs, pos_t)
    return y.reshape(orig_shape)


# speedup vs baseline: 2.0647x; 1.0039x over previous
import functools

import jax
import jax.numpy as jnp
from jax import lax
from jax.experimental import pallas as pl
from jax.experimental.pallas import tpu as pltpu
from jax.experimental.pallas import tpu_sc as plsc

NEXP = 8
BT = 512
NBP = 32
NW = 32
SUB = 32
HALF = 16


def _route_kernel(x_ref, wg_ref, pos_ref, w_ref, be_ref, act_ref):
    scoresT = lax.dot_general(wg_ref[...], x_ref[...],
                              (((1,), (1,)), ((), ())),
                              preferred_element_type=jnp.float32)
    e_, s_ = scoresT.shape
    sub = lax.broadcasted_iota(jnp.int32, (e_, s_), 0)
    v1 = jnp.max(scoresT, axis=0, keepdims=True)
    i1 = jnp.min(jnp.where(scoresT == v1, sub, e_), axis=0, keepdims=True)
    masked = jnp.where(sub == i1, -jnp.inf, scoresT)
    v2 = jnp.max(masked, axis=0, keepdims=True)
    i2 = jnp.min(jnp.where(masked == v2, sub, e_), axis=0, keepdims=True)
    w1 = 1.0 / (1.0 + jnp.exp(v2 - v1))
    w_ref[...] = jnp.concatenate([w1, 1.0 - w1], axis=1)

    oh = jnp.concatenate([sub == i1, sub == i2], axis=1).astype(jnp.int32)
    c = oh
    sh = 1
    while sh < 2 * s_:
        c = c + jnp.concatenate(
            [jnp.zeros((e_, sh), jnp.int32), c[:, :-sh]], axis=1)
        sh *= 2
    excl = c - oh
    counts = c[:, -1:]
    pc = ((counts + BT - 1) // BT) * BT
    oc = pc
    sh = 1
    while sh < e_:
        oc = oc + jnp.concatenate(
            [jnp.zeros((sh, 1), jnp.int32), oc[:-sh]], axis=0)
        sh *= 2
    off = oc - pc
    end = oc
    pos_ref[...] = jnp.sum(oh * (off + excl), axis=0, keepdims=True)

    starts = lax.broadcasted_iota(jnp.int32, (1, NBP), 1) * BT
    nbefore = jnp.sum((starts >= end).astype(jnp.int32), axis=0, keepdims=True)
    be_ref[...] = jnp.minimum(nbefore, e_ - 1)
    act_ref[...] = (starts < end[e_ - 1:e_, :]).astype(jnp.int32)


def _ffn_part(xs_ref, w1_ref, w3_ref, w2_ref, ws_ref):
    xb = xs_ref[...].astype(jnp.bfloat16)
    hid2 = w1_ref.shape[1]
    o = None
    c0 = 0
    while c0 < hid2:
        cw = min(512, hid2 - c0)
        g = lax.dot_general(xb, w1_ref[0, pl.ds(c0, cw), :],
                            (((1,), (1,)), ((), ())),
                            preferred_element_type=jnp.float32)
        u = lax.dot_general(xb, w3_ref[0, pl.ds(c0, cw), :],
                            (((1,), (1,)), ((), ())),
                            preferred_element_type=jnp.float32)
        a = (g * jax.nn.sigmoid(g) * u).astype(jnp.bfloat16)
        oc = lax.dot_general(a, w2_ref[0, :, pl.ds(c0, cw)],
                             (((1,), (1,)), ((), ())),
                             preferred_element_type=jnp.float32)
        o = oc if o is None else o + oc
        c0 += cw
    return o * ws_ref[:, 0:1]


def _ffn_kernel(be_ref, act_ref, xs_ref, w1_ref, w3_ref, w2_ref, ws_ref, o_ref):
    b = pl.program_id(0)

    @pl.when(act_ref[b] == 1)
    def _():
        o_ref[...] = _ffn_part(xs_ref, w1_ref, w3_ref, w2_ref, ws_ref)


def _ffn2_kernel(be_ref, act_ref, xs_ref, w1_ref, w3_ref, w2_ref, ws_ref,
                 prev_ref, o_ref):
    b = pl.program_id(0)

    @pl.when(act_ref[b] == 1)
    def _():
        o_ref[...] = prev_ref[...] + _ffn_part(
            xs_ref, w1_ref, w3_ref, w2_ref, ws_ref)


def _make_dispatch(s, dim, tpad):
    nsub = (2 * s // NW) // SUB
    mesh = plsc.VectorSubcoreMesh(core_axis_name="c", subcore_axis_name="s")

    apw = nsub * SUB

    @functools.partial(
        pl.kernel,
        out_type=(
            jax.ShapeDtypeStruct((tpad, dim), jnp.float32),
            jax.ShapeDtypeStruct((tpad, 128), jnp.float32),
        ),
        mesh=mesh,
        scratch_types=[
            pltpu.VMEM((nsub, SUB), jnp.int32),
            pltpu.VMEM((apw,), jnp.int32),
            pltpu.VMEM((SUB, dim), jnp.float32),
            pltpu.VMEM((SUB, dim), jnp.float32),
            pltpu.VMEM((apw, 128), jnp.float32),
            pltpu.SemaphoreType.DMA,
            pltpu.SemaphoreType.DMA,
            pltpu.SemaphoreType.DMA,
            pltpu.SemaphoreType.DMA,
            pltpu.SemaphoreType.DMA,
        ],
    )
    def dispatch(x_hbm, pos_hbm, posf_hbm, wrep_hbm, xs_hbm, ws_hbm,
                 idx_v, idxf_v, r0, r1, wbig,
                 semr0, semr1, sems0, sems1, semw):
        wid = lax.axis_index("s") * 2 + lax.axis_index("c")
        base = wid * apw
        tok_base = lax.rem(base, s)
        pltpu.sync_copy(pos_hbm.at[wid], idx_v)
        pltpu.sync_copy(posf_hbm.at[wid], idxf_v)
        pltpu.sync_copy(wrep_hbm.at[pl.ds(base, apw)], wbig)
        cw = pltpu.async_copy(wbig, ws_hbm.at[idxf_v], semw)
        rbufs = (r0, r1)
        rsems = (semr0, semr1)
        ssems = (sems0, sems1)
        pend_r = [None, None]
        pend_s = [None, None]

        def startr(i):
            p = i % 2
            if pend_s[p] is not None:
                pend_s[p].wait()
                pend_s[p] = None
            pend_r[p] = pltpu.async_copy(
                x_hbm.at[pl.ds(tok_base + i * SUB, SUB)], rbufs[p], rsems[p])

        startr(0)
        for i in range(nsub):
            p = i % 2
            if i + 1 < nsub:
                startr(i + 1)
            pend_r[p].wait()
            pend_s[p] = pltpu.async_copy(
                rbufs[p], xs_hbm.at[idx_v.at[i]], ssems[p])
        for p in (0, 1):
            if pend_s[p] is not None:
                pend_s[p].wait()
        cw.wait()

    return dispatch


def _make_combine(s, dim, tpad):
    tpt = s // NW
    nh = tpt // HALF
    nc = dim // 16
    mesh = plsc.VectorSubcoreMesh(core_axis_name="c", subcore_axis_name="s")

    @functools.partial(
        pl.kernel,
        out_type=jax.ShapeDtypeStruct((s, dim), jnp.float32),
        mesh=mesh,
        scratch_types=[
            pltpu.VMEM((2 * nh, HALF), jnp.int32),
            pltpu.VMEM((HALF, dim), jnp.float32),
            pltpu.VMEM((HALF, dim), jnp.float32),
            pltpu.VMEM((HALF, dim), jnp.float32),
            pltpu.VMEM((HALF, dim), jnp.float32),
            pltpu.SemaphoreType.DMA,
            pltpu.SemaphoreType.DMA,
            pltpu.SemaphoreType.DMA,
            pltpu.SemaphoreType.DMA,
        ],
    )
    def combine(out_hbm, pos_hbm, y_hbm, idx_v,
                ba0, bb0, ba1, bb1, semg0, semg1, semw0, semw1):
        wid = lax.axis_index("s") * 2 + lax.axis_index("c")
        tbase = wid * tpt
        pltpu.sync_copy(pos_hbm.at[wid], idx_v)
        bufs_a = (ba0, ba1)
        bufs_b = (bb0, bb1)
        gsems = (semg0, semg1)
        wsems = (semw0, semw1)
        pend_g = [None, None]
        pend_w = [None, None]

        def startg(h):
            p = h % 2
            if pend_w[p] is not None:
                pend_w[p].wait()
                pend_w[p] = None
            pend_g[p] = (
                pltpu.async_copy(out_hbm.at[idx_v.at[h]], bufs_a[p], gsems[p]),
                pltpu.async_copy(out_hbm.at[idx_v.at[nh + h]], bufs_b[p],
                                 gsems[p]))

        startg(0)
        for h in range(nh):
            p = h % 2
            if h + 1 < nh:
                startg(h + 1)
            ca, cb = pend_g[p]
            ca.wait()
            cb.wait()
            buf_a = bufs_a[p]
            buf_b = bufs_b[p]

            def row_body(r, _):
                def col_body(cc, _):
                    a = buf_a[r, pl.ds(cc * 16, 16)]
                    b = buf_b[r, pl.ds(cc * 16, 16)]
                    buf_a[r, pl.ds(cc * 16, 16)] = a + b
                    return 0

                lax.fori_loop(0, nc, col_body, 0, unroll=8)
                return 0

            lax.fori_loop(0, HALF, row_body, 0)
            pend_w[p] = pltpu.async_copy(
                buf_a, y_hbm.at[pl.ds(tbase + h * HALF, HALF)], wsems[p])
        for p in (0, 1):
            if pend_w[p] is not None:
                pend_w[p].wait()

    return combine


def kernel(x, Wg, W1, W3, W2):
    orig_shape = x.shape
    dim = orig_shape[-1]
    xf = x.reshape(-1, dim)
    s = xf.shape[0]
    nexp, hid, _ = W1.shape
    tpad = ((2 * s + NEXP * (BT - 1) + BT - 1) // BT) * BT
    nb = tpad // BT

    pos, w, be, act = pl.pallas_call(
        _route_kernel,
        out_shape=[
            jax.ShapeDtypeStruct((1, 2 * s), jnp.int32),
            jax.ShapeDtypeStruct((1, 2 * s), jnp.float32),
            jax.ShapeDtypeStruct((1, NBP), jnp.int32),
            jax.ShapeDtypeStruct((1, NBP), jnp.int32),
        ],
    )(xf, Wg)

    nsub = (2 * s // NW) // SUB
    wrep = jnp.broadcast_to(w.reshape(2 * s, 1), (2 * s, 128))
    xs, ws = _make_dispatch(s, dim, tpad)(
        xf, pos.reshape(NW, nsub, SUB), pos.reshape(NW, nsub * SUB), wrep)

    hid2 = hid // 2
    common_specs = [
        pl.BlockSpec((BT, dim), lambda b, be, act: (b, 0)),
    ]

    def wspecs(hh):
        return [
            pl.BlockSpec((1, hid2, dim), lambda b, be, act: (be[b], hh, 0)),
            pl.BlockSpec((1, hid2, dim), lambda b, be, act: (be[b], hh, 0)),
            pl.BlockSpec((1, dim, hid2), lambda b, be, act: (be[b], 0, hh)),
            pl.BlockSpec((BT, 128), lambda b, be, act: (b, 0)),
        ]

    outs0 = pl.pallas_call(
        _ffn_kernel,
        grid_spec=pltpu.PrefetchScalarGridSpec(
            num_scalar_prefetch=2,
            grid=(nb,),
            in_specs=common_specs + wspecs(0),
            out_specs=pl.BlockSpec((BT, dim), lambda b, be, act: (b, 0)),
        ),
        out_shape=jax.ShapeDtypeStruct((tpad, dim), jnp.float32),
    )(be.reshape(NBP), act.reshape(NBP), xs, W1, W3, W2, ws)

    outs = pl.pallas_call(
        _ffn2_kernel,
        grid_spec=pltpu.PrefetchScalarGridSpec(
            num_scalar_prefetch=2,
            grid=(nb,),
            in_specs=common_specs + wspecs(1)
            + [pl.BlockSpec((BT, dim), lambda b, be, act: (b, 0))],
            out_specs=pl.BlockSpec((BT, dim), lambda b, be, act: (b, 0)),
        ),
        out_shape=jax.ShapeDtypeStruct((tpad, dim), jnp.float32),
        input_output_aliases={7: 0},
    )(be.reshape(NBP), act.reshape(NBP), xs, W1, W3, W2, ws, outs0)

    tpt = s // NW
    nh = tpt // HALF
    p0 = pos[0, :s].reshape(NW, nh, HALF)
    p1 = pos[0, s:].reshape(NW, nh, HALF)
    pos_t = jnp.concatenate([p0, p1], axis=1)

    y = _make_combine(s, dim, tpad)(outs, pos_t)
    return y.reshape(orig_shape)
